# Initial kernel scaffold; baseline (speedup 1.0000x reference)
import jax
import jax.numpy as jnp
from jax.experimental import pallas as pl


def kernel(x_input, edge_index, batch, W0, b0, Wc1, bc1, Wc2, bc2, W1, b1, W2, b2, W3, b3):
    raise NotImplementedError("write your pallas kernel here")


# speedup vs baseline: 14.4050x; 14.4050x over previous
import functools

import jax
import jax.numpy as jnp
from jax import lax
from jax.experimental import pallas as pl
from jax.experimental.pallas import tpu as pltpu
from jax.experimental.pallas import tpu_sc as plsc

NC = 2
NS = 16
C = 128
BLK = 1024


def _round_up(a, b):
    return (a + b - 1) // b * b


def _make_deg_kernel(NP, K):
    mesh = plsc.VectorSubcoreMesh(
        core_axis_name="c", subcore_axis_name="s", num_cores=NC,
        num_subcores=NS)
    R = NP // NS

    @functools.partial(
        pl.kernel,
        out_type=jax.ShapeDtypeStruct((NC, NP, 1), jnp.float32),
        mesh=mesh,
        scratch_types=[
            pltpu.VMEM((K, C), jnp.int32),
            pltpu.VMEM((C, 1), jnp.float32),
            pltpu.VMEM_SHARED((NP, 1), jnp.float32),
        ],
        name="sg_deg",
    )
    def deg_kernel(dst3, ones_col, zeros_col, out, idx_v, ones_v, acc):
        cid = lax.axis_index("c")
        sid = lax.axis_index("s")
        w = cid * NS + sid
        for t in range(R // C):
            pltpu.sync_copy(zeros_col, acc.at[pl.ds(sid * R + t * C, C)])
        pltpu.sync_copy(ones_col, ones_v)
        pltpu.sync_copy(dst3.at[w], idx_v)
        plsc.subcore_barrier()

        def body(j, carry):
            pltpu.sync_copy(ones_v, acc.at[idx_v.at[j]], add=True)
            return carry

        lax.fori_loop(0, K, body, 0)
        plsc.subcore_barrier()
        pltpu.sync_copy(acc.at[pl.ds(sid * R, R)],
                        out.at[cid, pl.ds(sid * R, R)])

    return deg_kernel


def _make_prop_kernel(NP, K, F):
    mesh = plsc.VectorSubcoreMesh(
        core_axis_name="c", subcore_axis_name="s", num_cores=NC,
        num_subcores=NS)
    R = NP // NS

    @functools.partial(
        pl.kernel,
        out_type=jax.ShapeDtypeStruct((NC, NP, F), jnp.float32),
        mesh=mesh,
        scratch_types=[
            pltpu.VMEM((K, C), jnp.int32),
            pltpu.VMEM((K, C), jnp.int32),
            pltpu.VMEM((C, F), jnp.float32),
            pltpu.VMEM_SHARED((NP, F), jnp.float32),
            pltpu.SemaphoreType.DMA,
        ],
        name="sg_prop",
    )
    def prop_kernel(y_hbm, src3, dst3, zeros_rows, out, si_v, di_v, buf,
                    acc, sem):
        cid = lax.axis_index("c")
        sid = lax.axis_index("s")
        w = cid * NS + sid
        for t in range(R // C):
            pltpu.sync_copy(zeros_rows, acc.at[pl.ds(sid * R + t * C, C)])
        pltpu.sync_copy(src3.at[w], si_v)
        pltpu.sync_copy(dst3.at[w], di_v)
        plsc.subcore_barrier()

        def body(j, carry):
            pltpu.async_copy(y_hbm.at[si_v.at[j]], buf, sem).wait()
            pltpu.sync_copy(buf, acc.at[di_v.at[j]], add=True)
            return carry

        lax.fori_loop(0, K, body, 0)
        plsc.subcore_barrier()
        for t in range(R // C):
            pltpu.sync_copy(acc.at[pl.ds(sid * R + t * C, C)],
                            out.at[cid, pl.ds(sid * R + t * C, C)])

    return prop_kernel


def _relu(x):
    return jnp.maximum(x, 0.0)


def _lin0_body(deg_ref, x_ref, w_ref, b_ref, y_ref, dis_ref, dis2_ref):
    deg = deg_ref[0] + deg_ref[1]
    dis = jnp.where(deg > 0, lax.rsqrt(jnp.maximum(deg, 1e-12)), 0.0)
    h = _relu(jnp.dot(x_ref[...], w_ref[...],
                      preferred_element_type=jnp.float32) + b_ref[...])
    y_ref[...] = dis * h
    dis_ref[...] = dis
    dis2_ref[...] = dis * dis


def _comb_body(p_ref, dis2_ref, y_ref):
    y_ref[...] = dis2_ref[...] * (p_ref[0] + p_ref[1])


def _convlin_body(p_ref, dis_ref, w_ref, b_ref, y_ref):
    h = dis_ref[...] * (p_ref[0] + p_ref[1])
    x = _relu(jnp.dot(h, w_ref[...],
                      preferred_element_type=jnp.float32) + b_ref[...])
    y_ref[...] = dis_ref[...] * x


def _final_body(p_ref, dis_ref, wc_ref, bc_ref, w1_ref, b1_ref, w2_ref,
                b2_ref, w3_ref, b3_ref, o_ref):
    h = dis_ref[...] * (p_ref[0] + p_ref[1])
    x = _relu(jnp.dot(h, wc_ref[...],
                      preferred_element_type=jnp.float32) + bc_ref[...])
    x = _relu(jnp.dot(x, w1_ref[...],
                      preferred_element_type=jnp.float32) + b1_ref[...])
    x = _relu(jnp.dot(x, w2_ref[...],
                      preferred_element_type=jnp.float32) + b2_ref[...])
    o_ref[...] = _relu(jnp.dot(x, w3_ref[...],
                               preferred_element_type=jnp.float32)
                       + b3_ref[...])


def _col_spec(i_map=lambda i: (i, 0)):
    return pl.BlockSpec((BLK, 1), i_map)


def _full(shape):
    return pl.BlockSpec(shape, lambda i: tuple(0 for _ in shape))


def kernel(x_input, edge_index, batch, W0, b0, Wc1, bc1, Wc2, bc2, W1, b1,
           W2, b2, W3, b3):
    N, F = x_input.shape
    H = W0.shape[1]
    E = edge_index.shape[1]
    NP = _round_up(N, BLK)
    PADR = NP - N if NP > N else NP
    EL = E + N
    W = NC * NS
    K = _round_up(_round_up(EL, W * C) // (W * C), 2)
    EP = W * K * C

    idx_n = jnp.arange(N, dtype=jnp.int32)
    pad_i = jnp.arange(EP - EL, dtype=jnp.int32)
    src_p = jnp.concatenate([edge_index[0], idx_n, pad_i % N])
    dst_p = jnp.concatenate([edge_index[1], idx_n, N + pad_i % PADR])
    src3 = src_p.reshape(W, K, C)
    dst3 = dst_p.reshape(W, K, C)

    xp = jnp.pad(x_input, ((0, NP - N), (0, 0)))
    ones_col = jnp.ones((C, 1), jnp.float32)
    zeros_col = jnp.zeros((C, 1), jnp.float32)
    zeros_rows = jnp.zeros((C, F), jnp.float32)
    b0r, bc1r, bc2r, b1r, b2r = (v.reshape(1, H)
                                 for v in (b0, bc1, bc2, b1, b2))
    b3r = b3.reshape(1, 1)

    deg_k = _make_deg_kernel(NP, K)
    prop_k = _make_prop_kernel(NP, K, F)

    grid = (NP // BLK,)
    row_spec = pl.BlockSpec((BLK, F), lambda i: (i, 0))
    p_spec = pl.BlockSpec((NC, BLK, F), lambda i: (0, i, 0))

    deg_parts = deg_k(dst3, ones_col, zeros_col)

    lin0 = pl.pallas_call(
        _lin0_body,
        grid=grid,
        in_specs=[pl.BlockSpec((NC, BLK, 1), lambda i: (0, i, 0)),
                  row_spec, _full((F, H)), _full((1, H))],
        out_specs=[row_spec, _col_spec(), _col_spec()],
        out_shape=[jax.ShapeDtypeStruct((NP, H), jnp.float32),
                   jax.ShapeDtypeStruct((NP, 1), jnp.float32),
                   jax.ShapeDtypeStruct((NP, 1), jnp.float32)],
    )
    y, dis, dis2 = lin0(deg_parts, xp, W0, b0r)

    combine = pl.pallas_call(
        _comb_body,
        grid=grid,
        in_specs=[p_spec, _col_spec()],
        out_specs=row_spec,
        out_shape=jax.ShapeDtypeStruct((NP, H), jnp.float32),
    )
    convlin = pl.pallas_call(
        _convlin_body,
        grid=grid,
        in_specs=[p_spec, _col_spec(), _full((H, H)), _full((1, H))],
        out_specs=row_spec,
        out_shape=jax.ShapeDtypeStruct((NP, H), jnp.float32),
    )
    final = pl.pallas_call(
        _final_body,
        grid=grid,
        in_specs=[p_spec, _col_spec(), _full((H, H)), _full((1, H)),
                  _full((H, H)), _full((1, H)), _full((H, H)), _full((1, H)),
                  _full((H, 1)), _full((1, 1))],
        out_specs=_col_spec(),
        out_shape=jax.ShapeDtypeStruct((NP, 1), jnp.float32),
    )

    for r in range(2):
        parts = prop_k(y, src3, dst3, zeros_rows)
        y = combine(parts, dis2)
    parts = prop_k(y, src3, dst3, zeros_rows)
    y = convlin(parts, dis, Wc1, bc1r)
    for r in range(2):
        parts = prop_k(y, src3, dst3, zeros_rows)
        y = combine(parts, dis2)
    parts = prop_k(y, src3, dst3, zeros_rows)
    out = final(parts, dis, Wc2, bc2r, W1, b1r, W2, b2r, W3, b3r)
    return out[:N]
